# baseline (speedup 1.0000x reference)
import jax
import jax.numpy as jnp
from jax.experimental import pallas as pl
from jax.experimental.pallas import tpu as pltpu

_BK = 64
_NBUF = 32


def _body(pos_ref, x_hbm, o_hbm, xbuf, obuf, insem, outsem):
    P, B, D = x_hbm.shape
    per_plane = B // _BK
    nchunks = P * per_plane

    def in_copy(chunk, slot):
        p = jax.lax.div(chunk, per_plane)
        i = jax.lax.rem(chunk, per_plane)
        return pltpu.make_async_copy(
            x_hbm.at[p, pl.ds(i * _BK, _BK)], xbuf.at[slot], insem.at[slot])

    def out_copy(chunk, slot):
        p = jax.lax.div(chunk, per_plane)
        i = jax.lax.rem(chunk, per_plane)
        return pltpu.make_async_copy(
            obuf.at[slot], o_hbm.at[p, pl.ds(i * _BK, _BK)], outsem.at[slot])

    for k in range(_NBUF):
        in_copy(jnp.int32(k), k).start()

    def step(c, carry):
        slot = jax.lax.rem(c, _NBUF)
        p = jax.lax.div(c, per_plane)
        in_copy(c, slot).wait()

        @pl.when(c >= _NBUF)
        def _():
            out_copy(c - _NBUF, slot).wait()

        obuf[slot] = xbuf[slot] + pos_ref[pl.ds(p, 1)]
        out_copy(c, slot).start()

        @pl.when(c + _NBUF < nchunks)
        def _():
            in_copy(c + _NBUF, slot).start()

        return carry

    jax.lax.fori_loop(0, nchunks, step, 0)

    for k in range(_NBUF):
        c = nchunks - _NBUF + k
        out_copy(jnp.int32(c), c % _NBUF).wait()


def kernel(patch, pos_emb):
    B, P, D = patch.shape
    xt = jnp.transpose(patch, (1, 0, 2))
    out = pl.pallas_call(
        _body,
        in_specs=[
            pl.BlockSpec((P, D), lambda: (0, 0)),
            pl.BlockSpec(memory_space=pl.ANY),
        ],
        out_specs=pl.BlockSpec(memory_space=pl.ANY),
        out_shape=jax.ShapeDtypeStruct((P, B, D), patch.dtype),
        scratch_shapes=[
            pltpu.VMEM((_NBUF, _BK, D), patch.dtype),
            pltpu.VMEM((_NBUF, _BK, D), patch.dtype),
            pltpu.SemaphoreType.DMA((_NBUF,)),
            pltpu.SemaphoreType.DMA((_NBUF,)),
        ],
    )(pos_emb, xt)
    return jnp.transpose(out, (1, 0, 2))


# speedup vs baseline: 1.0035x; 1.0035x over previous
import jax
import jax.numpy as jnp
from jax.experimental import pallas as pl
from jax.experimental.pallas import tpu as pltpu

_BK = 1024
_NBUF = 6


def _body(pos_ref, x_hbm, o_hbm, xbuf, obuf, insem, outsem):
    P, B, D = x_hbm.shape
    per_plane = B // _BK
    nchunks = P * per_plane

    def in_copy(chunk, slot):
        p = jax.lax.div(chunk, per_plane)
        i = jax.lax.rem(chunk, per_plane)
        return pltpu.make_async_copy(
            x_hbm.at[p, pl.ds(i * _BK, _BK)], xbuf.at[slot], insem.at[slot])

    def out_copy(chunk, slot):
        p = jax.lax.div(chunk, per_plane)
        i = jax.lax.rem(chunk, per_plane)
        return pltpu.make_async_copy(
            obuf.at[slot], o_hbm.at[p, pl.ds(i * _BK, _BK)], outsem.at[slot])

    for k in range(_NBUF):
        in_copy(jnp.int32(k), k).start()

    def step(c, carry):
        slot = jax.lax.rem(c, _NBUF)
        p = jax.lax.div(c, per_plane)
        in_copy(c, slot).wait()

        @pl.when(c >= _NBUF)
        def _():
            out_copy(c - _NBUF, slot).wait()

        obuf[slot] = xbuf[slot] + pos_ref[pl.ds(p, 1)]
        out_copy(c, slot).start()

        @pl.when(c + _NBUF < nchunks)
        def _():
            in_copy(c + _NBUF, slot).start()

        return carry

    jax.lax.fori_loop(0, nchunks, step, 0)

    for k in range(_NBUF):
        c = nchunks - _NBUF + k
        out_copy(jnp.int32(c), c % _NBUF).wait()


def kernel(patch, pos_emb):
    B, P, D = patch.shape
    xt = jnp.transpose(patch, (1, 0, 2))
    out = pl.pallas_call(
        _body,
        in_specs=[
            pl.BlockSpec((P, D), lambda: (0, 0)),
            pl.BlockSpec(memory_space=pl.ANY),
        ],
        out_specs=pl.BlockSpec(memory_space=pl.ANY),
        out_shape=jax.ShapeDtypeStruct((P, B, D), patch.dtype),
        scratch_shapes=[
            pltpu.VMEM((_NBUF, _BK, D), patch.dtype),
            pltpu.VMEM((_NBUF, _BK, D), patch.dtype),
            pltpu.SemaphoreType.DMA((_NBUF,)),
            pltpu.SemaphoreType.DMA((_NBUF,)),
        ],
    )(pos_emb, xt)
    return jnp.transpose(out, (1, 0, 2))


# speedup vs baseline: 1.0054x; 1.0018x over previous
import jax
import jax.numpy as jnp
from jax.experimental import pallas as pl
from jax.experimental.pallas import tpu as pltpu

_BK = 2048
_NBUF = 3


def _body(pos_ref, x_hbm, o_hbm, xbuf, obuf, insem, outsem):
    P, B, D = x_hbm.shape
    per_plane = B // _BK
    nchunks = P * per_plane

    def in_copy(chunk, slot):
        p = jax.lax.div(chunk, per_plane)
        i = jax.lax.rem(chunk, per_plane)
        return pltpu.make_async_copy(
            x_hbm.at[p, pl.ds(i * _BK, _BK)], xbuf.at[slot], insem.at[slot])

    def out_copy(chunk, slot):
        p = jax.lax.div(chunk, per_plane)
        i = jax.lax.rem(chunk, per_plane)
        return pltpu.make_async_copy(
            obuf.at[slot], o_hbm.at[p, pl.ds(i * _BK, _BK)], outsem.at[slot])

    for k in range(_NBUF):
        in_copy(jnp.int32(k), k).start()

    def step(c, carry):
        slot = jax.lax.rem(c, _NBUF)
        p = jax.lax.div(c, per_plane)
        in_copy(c, slot).wait()

        @pl.when(c >= _NBUF)
        def _():
            out_copy(c - _NBUF, slot).wait()

        obuf[slot] = xbuf[slot] + pos_ref[pl.ds(p, 1)]
        out_copy(c, slot).start()

        @pl.when(c + _NBUF < nchunks)
        def _():
            in_copy(c + _NBUF, slot).start()

        return carry

    jax.lax.fori_loop(0, nchunks, step, 0)

    for k in range(_NBUF):
        c = nchunks - _NBUF + k
        out_copy(jnp.int32(c), c % _NBUF).wait()


def kernel(patch, pos_emb):
    B, P, D = patch.shape
    xt = jnp.transpose(patch, (1, 0, 2))
    out = pl.pallas_call(
        _body,
        in_specs=[
            pl.BlockSpec((P, D), lambda: (0, 0)),
            pl.BlockSpec(memory_space=pl.ANY),
        ],
        out_specs=pl.BlockSpec(memory_space=pl.ANY),
        out_shape=jax.ShapeDtypeStruct((P, B, D), patch.dtype),
        scratch_shapes=[
            pltpu.VMEM((_NBUF, _BK, D), patch.dtype),
            pltpu.VMEM((_NBUF, _BK, D), patch.dtype),
            pltpu.SemaphoreType.DMA((_NBUF,)),
            pltpu.SemaphoreType.DMA((_NBUF,)),
        ],
    )(pos_emb, xt)
    return jnp.transpose(out, (1, 0, 2))
